# baseline (speedup 1.0000x reference)
import functools

import jax
import jax.numpy as jnp
from jax import lax
from jax.experimental import pallas as pl
from jax.experimental.pallas import tpu as pltpu
from jax.experimental.pallas import tpu_sc as plsc

NB = 320000
NA = 10000
NA_PAD = 10240
H = 128
MAXNB = 32
NMOL = 500
APM = 20

NC = 2
NS = 16
NW = NC * NS


_SC_MESH = dict(core_axis_name="c", subcore_axis_name="s")


def _agg_body(relu_rows, msg_hbm, a2b_hbm, w_hbm, out_hbm,
              idx_v, rows_v, w_v, out_big, *sems):
    c = lax.axis_index("c")
    sid = lax.axis_index("s")
    a_sc0 = 432
    a_sc1 = NA_PAD // NS - a_sc0
    base = jnp.where(c == 0, sid * a_sc0, NS * a_sc0 + sid * a_sc1)
    atoms_per_w = jnp.where(c == 0, a_sc0, a_sc1)
    steps = atoms_per_w // 4
    sem_r = sems[0:2]
    sem_w = sems[2:4]

    def issue(j, b):
        off = (base + j * 4) * MAXNB
        pltpu.sync_copy(a2b_hbm.at[pl.ds(off, 128)], idx_v.at[b])
        pltpu.async_copy(msg_hbm.at[idx_v.at[b]], rows_v.at[b], sem_r[b])
        pltpu.async_copy(w_hbm.at[idx_v.at[b]], w_v.at[b, pl.ds(0, 128)],
                         sem_w[b])

    def compute(j, b):
        pltpu.make_async_copy(msg_hbm.at[idx_v.at[b]], rows_v.at[b],
                              sem_r[b]).wait()
        pltpu.make_async_copy(w_hbm.at[idx_v.at[b]],
                              w_v.at[b, pl.ds(0, 128)], sem_w[b]).wait()
        for a in range(4):
            wv0 = w_v[b, pl.ds(a * MAXNB, 16)]
            wv1 = w_v[b, pl.ds(a * MAXNB + 16, 16)]
            accs = [jnp.zeros((16,), jnp.float32) for _ in range(8)]
            for k in range(MAXNB):
                w = (wv0 if k < 16 else wv1)[k % 16]
                r = a * MAXNB + k
                for v in range(8):
                    row = rows_v[b, r, pl.ds(v * 16, 16)]
                    if relu_rows:
                        row = jnp.maximum(row, 0.0)
                    accs[v] = accs[v] + w * row
            for v in range(8):
                out_big[j * 4 + a, pl.ds(v * 16, 16)] = accs[v]

    issue(0, 0)

    def quad(i, _):
        for t in range(4):
            j = 4 * i + t

            @pl.when(j + 1 < steps)
            def _():
                issue(j + 1, (t + 1) % 2)
            compute(j, t % 2)
        return 0

    lax.fori_loop(0, steps // 4, quad, 0)

    @pl.when(c == 0)
    def _():
        pltpu.sync_copy(out_big, out_hbm.at[pl.ds(base, a_sc0)])

    @pl.when(c == 1)
    def _():
        pltpu.sync_copy(out_big.at[pl.ds(0, a_sc1)],
                        out_hbm.at[pl.ds(base, a_sc1)])


def _make_agg(relu_rows):
    return functools.partial(
        pl.kernel,
        functools.partial(_agg_body, relu_rows),
        out_type=jax.ShapeDtypeStruct((NA_PAD, H), jnp.float32),
        mesh=plsc.VectorSubcoreMesh(**_SC_MESH),
        scratch_types=[
            pltpu.VMEM((2, 128), jnp.int32),
            pltpu.VMEM((2, 128, H), jnp.float32),
            pltpu.VMEM((2, 144), jnp.float32),
            pltpu.VMEM((432, H), jnp.float32),
        ] + [pltpu.SemaphoreType.DMA] * 4,
    )()


_agg_relu = _make_agg(True)
_agg_plain = _make_agg(False)

_DSTEP = 80
_DBUF = 4


def _diff_body(relu_rows, amsg_hbm, msg_hbm, b2a_hbm, b2revb_hbm, out_hbm,
               idxa, idxr, rr_v, out_v, *sems):
    sa = (sems[0:4], sems[4:8])
    sr = (sems[8:12], sems[12:16])
    so = sems[16:20]
    wid = lax.axis_index("s") * NC + lax.axis_index("c")
    bonds_per_w = NB // NW
    steps = bonds_per_w // _DSTEP

    hs = _DSTEP // 2

    def issue(j, t, drain_store):
        if drain_store:
            pltpu.make_async_copy(out_v.at[t], out_hbm.at[pl.ds(0, _DSTEP)],
                                  so[t]).wait()
        b0 = wid * bonds_per_w + j * _DSTEP
        pltpu.sync_copy(b2a_hbm.at[pl.ds(b0, _DSTEP)], idxa.at[t])
        pltpu.sync_copy(b2revb_hbm.at[pl.ds(b0, _DSTEP)], idxr.at[t])
        for q in range(2):
            pltpu.async_copy(amsg_hbm.at[idxa.at[t, pl.ds(q * hs, hs)]],
                             out_v.at[t, pl.ds(q * hs, hs)], sa[q][t])
            pltpu.async_copy(msg_hbm.at[idxr.at[t, pl.ds(q * hs, hs)]],
                             rr_v.at[t, pl.ds(q * hs, hs)], sr[q][t])

    def compute(j, t):
        for q in range(2):
            pltpu.make_async_copy(amsg_hbm.at[idxa.at[t, pl.ds(q * hs, hs)]],
                                  out_v.at[t, pl.ds(q * hs, hs)],
                                  sa[q][t]).wait()
            pltpu.make_async_copy(msg_hbm.at[idxr.at[t, pl.ds(q * hs, hs)]],
                                  rr_v.at[t, pl.ds(q * hs, hs)],
                                  sr[q][t]).wait()

        def row(r, _):
            for v in range(8):
                sl = pl.ds(v * 16, 16)
                rr = rr_v[t, r, sl]
                if relu_rows:
                    rr = jnp.maximum(rr, 0.0)
                plsc.addupdate(out_v.at[t, r, sl], -rr)
            return 0
        lax.fori_loop(0, _DSTEP, row, 0, unroll=4)
        b0 = wid * bonds_per_w + j * _DSTEP
        pltpu.async_copy(out_v.at[t], out_hbm.at[pl.ds(b0, _DSTEP)], so[t])

    issue(0, 0, drain_store=False)

    def quad(i, _):
        for t in range(_DBUF):
            j = 4 * i + t
            nt = (t + 1) % _DBUF

            @pl.when(j + 1 >= _DBUF)
            def _():
                issue(j + 1, nt, drain_store=True)

            @pl.when(j + 1 < _DBUF)
            def _():
                issue(j + 1, nt, drain_store=False)
            compute(j, t)
        return 0

    lax.fori_loop(0, (steps - 1) // _DBUF, quad, 0)
    compute(steps - 1, 0)
    for t in range(_DBUF):
        pltpu.make_async_copy(out_v.at[t], out_hbm.at[pl.ds(0, _DSTEP)],
                              so[t]).wait()


def _make_diff(relu_rows):
    return functools.partial(
        pl.kernel,
        functools.partial(_diff_body, relu_rows),
        out_type=jax.ShapeDtypeStruct((NB, H), jnp.float32),
        mesh=plsc.VectorSubcoreMesh(**_SC_MESH),
        scratch_types=[
            pltpu.VMEM((_DBUF, _DSTEP), jnp.int32),
            pltpu.VMEM((_DBUF, _DSTEP), jnp.int32),
            pltpu.VMEM((_DBUF, _DSTEP, H), jnp.float32),
            pltpu.VMEM((_DBUF, _DSTEP, H), jnp.float32),
        ] + [pltpu.SemaphoreType.DMA] * 20,
    )()


_diff_relu = _make_diff(True)
_diff_plain = _make_diff(False)


_BM = 2000


def _proj_body(x_ref, w_ref, o_ref):
    o_ref[...] = jnp.dot(x_ref[...], w_ref[...],
                         preferred_element_type=jnp.float32)


def _proj(x, w):
    m, k = x.shape
    n = w.shape[1]
    return pl.pallas_call(
        _proj_body,
        grid=(m // _BM,),
        in_specs=[
            pl.BlockSpec((_BM, k), lambda i: (i, 0)),
            pl.BlockSpec((k, n), lambda i: (0, 0)),
        ],
        out_specs=pl.BlockSpec((_BM, n), lambda i: (i, 0)),
        out_shape=jax.ShapeDtypeStruct((m, n), jnp.float32),
    )(x, w)


def _projh_body(relu_x, x_ref, w_ref, o_ref):
    x = x_ref[...]
    if relu_x:
        x = jnp.maximum(x, 0.0)
    o_ref[...] = jnp.dot(x, w_ref[...], preferred_element_type=jnp.float32)


def _projh(x, w, relu_x=False, block=None):
    m = x.shape[0]
    bm = block or _BM
    return pl.pallas_call(
        functools.partial(_projh_body, relu_x),
        grid=(m // bm,),
        in_specs=[
            pl.BlockSpec((bm, H), lambda i: (i, 0)),
            pl.BlockSpec((H, H), lambda i: (0, 0)),
        ],
        out_specs=pl.BlockSpec((bm, H), lambda i: (i, 0)),
        out_shape=jax.ShapeDtypeStruct((m, H), jnp.float32),
    )(x, w)


def _resid_body(d_ref, inp_ref, o_ref):
    o_ref[...] = jnp.maximum(inp_ref[...] + d_ref[...], 0.0)


def _resid(d, inp):
    return pl.pallas_call(
        _resid_body,
        grid=(NB // _BM,),
        in_specs=[
            pl.BlockSpec((_BM, H), lambda i: (i, 0)),
            pl.BlockSpec((_BM, H), lambda i: (i, 0)),
        ],
        out_specs=pl.BlockSpec((_BM, H), lambda i: (i, 0)),
        out_shape=jax.ShapeDtypeStruct((NB, H), jnp.float32),
    )(d, inp)


def _update_body(d_ref, inp_ref, w_ref, o_ref):
    o_ref[...] = jnp.maximum(
        inp_ref[...] + jnp.dot(d_ref[...], w_ref[...],
                               preferred_element_type=jnp.float32), 0.0)


def _update(d, inp, w_h):
    return pl.pallas_call(
        _update_body,
        grid=(NB // _BM,),
        in_specs=[
            pl.BlockSpec((_BM, H), lambda i: (i, 0)),
            pl.BlockSpec((_BM, H), lambda i: (i, 0)),
            pl.BlockSpec((H, H), lambda i: (0, 0)),
        ],
        out_specs=pl.BlockSpec((_BM, H), lambda i: (i, 0)),
        out_shape=jax.ShapeDtypeStruct((NB, H), jnp.float32),
    )(d, inp, w_h)


def _atomout_body(fa_ref, am_ref, w1_ref, w2_ref, b_ref, o_ref):
    acc = jnp.dot(fa_ref[...], w1_ref[...], preferred_element_type=jnp.float32)
    acc += jnp.dot(am_ref[...], w2_ref[...], preferred_element_type=jnp.float32)
    o_ref[...] = jnp.maximum(acc + b_ref[...], 0.0)


def _atom_out(f_atoms, a_msg_pad, w1, w2, b):
    return pl.pallas_call(
        _atomout_body,
        grid=(NA // 1000,),
        in_specs=[
            pl.BlockSpec((1000, H), lambda i: (i, 0)),
            pl.BlockSpec((1000, H), lambda i: (i, 0)),
            pl.BlockSpec((H, H), lambda i: (0, 0)),
            pl.BlockSpec((H, H), lambda i: (0, 0)),
            pl.BlockSpec((1, H), lambda i: (0, 0)),
        ],
        out_specs=pl.BlockSpec((1000, H), lambda i: (i, 0)),
        out_shape=jax.ShapeDtypeStruct((NA, H), jnp.float32),
    )(f_atoms, a_msg_pad, w1, w2, b)


def _readout_body(h2_ref, w_ref, deg_ref, o_ref):
    acc = w_ref[:, 0:1] * h2_ref[:, 0:H]
    wsum = w_ref[:, 0:1]
    for k in range(1, APM):
        acc += w_ref[:, k:k + 1] * h2_ref[:, k * H:(k + 1) * H]
        wsum += w_ref[:, k:k + 1]
    o_ref[...] = deg_ref[...] * acc / wsum


def _readout(h2, w_atoms2, deg2):
    return pl.pallas_call(
        _readout_body,
        in_specs=[
            pl.BlockSpec((NMOL, APM * H), lambda: (0, 0)),
            pl.BlockSpec((NMOL, APM), lambda: (0, 0)),
            pl.BlockSpec((NMOL, 1), lambda: (0, 0)),
        ],
        out_specs=pl.BlockSpec((NMOL, H), lambda: (0, 0)),
        out_shape=jax.ShapeDtypeStruct((NMOL, H), jnp.float32),
    )(h2, w_atoms2, deg2)


def kernel(f_atoms, f_bonds, w_atoms, w_bonds, degree_of_polym,
           a2b, b2a, b2revb, W_i, W_h, W_o_w, W_o_b):
    a2b_flat = jnp.concatenate(
        [a2b, jnp.zeros((NA_PAD - NA, MAXNB), jnp.int32)], axis=0).reshape(-1)
    wo1 = W_o_w[:H]
    wo2 = W_o_w[H:]
    bias2 = W_o_b.reshape(1, H)
    w_atoms2 = w_atoms.reshape(NMOL, APM)
    deg2 = degree_of_polym.reshape(NMOL, 1)

    inp = _proj(f_bonds, W_i)

    r1 = _projh(inp, W_h, relu_x=True)
    a1 = _agg_relu(inp, a2b_flat, w_bonds)
    A1 = _projh(a1, W_h, block=1024)
    d1 = _diff_plain(A1, r1, b2a, b2revb)
    msg2 = _resid(d1, inp)

    r2 = _projh(msg2, W_h)
    a2 = _agg_plain(msg2, a2b_flat, w_bonds)
    A2 = _projh(a2, W_h, block=1024)
    d2 = _diff_plain(A2, r2, b2a, b2revb)
    msg3 = _resid(d2, inp)

    a3 = _agg_plain(msg3, a2b_flat, w_bonds)
    h = _atom_out(f_atoms, a3[:NA], wo1, wo2, bias2)
    h2 = h.reshape(NMOL, APM * H)
    return _readout(h2, w_atoms2, deg2)


# speedup vs baseline: 1.1429x; 1.1429x over previous
import functools

import jax
import jax.numpy as jnp
from jax import lax
from jax.experimental import pallas as pl
from jax.experimental.pallas import tpu as pltpu
from jax.experimental.pallas import tpu_sc as plsc

NB = 320000
NA = 10000
NA_PAD = 10240
H = 128
MAXNB = 32
NMOL = 500
APM = 20

NC = 2
NS = 16
NW = NC * NS


_SC_MESH = dict(core_axis_name="c", subcore_axis_name="s")


def _agg_body(relu_rows, msg_hbm, a2b_hbm, w_hbm, out_hbm,
              idx_v, rows_v, w_v, out_big, *sems):
    c = lax.axis_index("c")
    sid = lax.axis_index("s")
    a_sc0 = 432
    a_sc1 = NA_PAD // NS - a_sc0
    base = jnp.where(c == 0, sid * a_sc0, NS * a_sc0 + sid * a_sc1)
    atoms_per_w = jnp.where(c == 0, a_sc0, a_sc1)
    steps = atoms_per_w // 4
    sem_r = sems[0:2]
    sem_w = sems[2:4]

    def issue(j, b):
        off = (base + j * 4) * MAXNB
        pltpu.sync_copy(a2b_hbm.at[pl.ds(off, 128)], idx_v.at[b])
        pltpu.async_copy(msg_hbm.at[idx_v.at[b]], rows_v.at[b], sem_r[b])
        pltpu.async_copy(w_hbm.at[idx_v.at[b]], w_v.at[b, pl.ds(0, 128)],
                         sem_w[b])

    def compute(j, b):
        pltpu.make_async_copy(msg_hbm.at[idx_v.at[b]], rows_v.at[b],
                              sem_r[b]).wait()
        pltpu.make_async_copy(w_hbm.at[idx_v.at[b]],
                              w_v.at[b, pl.ds(0, 128)], sem_w[b]).wait()
        for a in range(4):
            wv0 = w_v[b, pl.ds(a * MAXNB, 16)]
            wv1 = w_v[b, pl.ds(a * MAXNB + 16, 16)]
            accs = [jnp.zeros((16,), jnp.float32) for _ in range(8)]
            for k in range(MAXNB):
                w = (wv0 if k < 16 else wv1)[k % 16]
                r = a * MAXNB + k
                for v in range(8):
                    row = rows_v[b, r, pl.ds(v * 16, 16)]
                    if relu_rows:
                        row = jnp.maximum(row, 0.0)
                    accs[v] = accs[v] + w * row
            for v in range(8):
                out_big[j * 4 + a, pl.ds(v * 16, 16)] = accs[v]

    issue(0, 0)

    def quad(i, _):
        for t in range(4):
            j = 4 * i + t

            @pl.when(j + 1 < steps)
            def _():
                issue(j + 1, (t + 1) % 2)
            compute(j, t % 2)
        return 0

    lax.fori_loop(0, steps // 4, quad, 0)

    @pl.when(c == 0)
    def _():
        pltpu.sync_copy(out_big, out_hbm.at[pl.ds(base, a_sc0)])

    @pl.when(c == 1)
    def _():
        pltpu.sync_copy(out_big.at[pl.ds(0, a_sc1)],
                        out_hbm.at[pl.ds(base, a_sc1)])


def _make_agg(relu_rows):
    return functools.partial(
        pl.kernel,
        functools.partial(_agg_body, relu_rows),
        out_type=jax.ShapeDtypeStruct((NA_PAD, H), jnp.float32),
        mesh=plsc.VectorSubcoreMesh(**_SC_MESH),
        scratch_types=[
            pltpu.VMEM((2, 128), jnp.int32),
            pltpu.VMEM((2, 128, H), jnp.float32),
            pltpu.VMEM((2, 144), jnp.float32),
            pltpu.VMEM((432, H), jnp.float32),
        ] + [pltpu.SemaphoreType.DMA] * 4,
    )()


_agg_relu = _make_agg(True)
_agg_plain = _make_agg(False)

_DSTEP = 80
_DBUF = 4


def _diff_body(relu_rows, amsg_hbm, msg_hbm, b2a_hbm, b2revb_hbm, out_hbm,
               idxa, idxr, rr_v, out_v, *sems):
    sa = (sems[0:4], sems[4:8])
    sr = (sems[8:12], sems[12:16])
    so = sems[16:20]
    wid = lax.axis_index("s") * NC + lax.axis_index("c")
    bonds_per_w = NB // NW
    steps = bonds_per_w // _DSTEP

    hs = _DSTEP // 2

    def issue(j, t, drain_store):
        if drain_store:
            pltpu.make_async_copy(out_v.at[t], out_hbm.at[pl.ds(0, _DSTEP)],
                                  so[t]).wait()
        b0 = wid * bonds_per_w + j * _DSTEP
        pltpu.sync_copy(b2a_hbm.at[pl.ds(b0, _DSTEP)], idxa.at[t])
        pltpu.sync_copy(b2revb_hbm.at[pl.ds(b0, _DSTEP)], idxr.at[t])
        for q in range(2):
            pltpu.async_copy(amsg_hbm.at[idxa.at[t, pl.ds(q * hs, hs)]],
                             out_v.at[t, pl.ds(q * hs, hs)], sa[q][t])
            pltpu.async_copy(msg_hbm.at[idxr.at[t, pl.ds(q * hs, hs)]],
                             rr_v.at[t, pl.ds(q * hs, hs)], sr[q][t])

    def compute(j, t):
        for q in range(2):
            pltpu.make_async_copy(amsg_hbm.at[idxa.at[t, pl.ds(q * hs, hs)]],
                                  out_v.at[t, pl.ds(q * hs, hs)],
                                  sa[q][t]).wait()
            pltpu.make_async_copy(msg_hbm.at[idxr.at[t, pl.ds(q * hs, hs)]],
                                  rr_v.at[t, pl.ds(q * hs, hs)],
                                  sr[q][t]).wait()

        def row(r, _):
            for v in range(8):
                sl = pl.ds(v * 16, 16)
                rr = rr_v[t, r, sl]
                if relu_rows:
                    rr = jnp.maximum(rr, 0.0)
                plsc.addupdate(out_v.at[t, r, sl], -rr)
            return 0
        lax.fori_loop(0, _DSTEP, row, 0, unroll=4)
        b0 = wid * bonds_per_w + j * _DSTEP
        pltpu.async_copy(out_v.at[t], out_hbm.at[pl.ds(b0, _DSTEP)], so[t])

    issue(0, 0, drain_store=False)

    def quad(i, _):
        for t in range(_DBUF):
            j = 4 * i + t
            nt = (t + 1) % _DBUF

            @pl.when(j + 1 >= _DBUF)
            def _():
                issue(j + 1, nt, drain_store=True)

            @pl.when(j + 1 < _DBUF)
            def _():
                issue(j + 1, nt, drain_store=False)
            compute(j, t)
        return 0

    lax.fori_loop(0, (steps - 1) // _DBUF, quad, 0)
    compute(steps - 1, 0)
    for t in range(_DBUF):
        pltpu.make_async_copy(out_v.at[t], out_hbm.at[pl.ds(0, _DSTEP)],
                              so[t]).wait()


def _make_diff(relu_rows):
    return functools.partial(
        pl.kernel,
        functools.partial(_diff_body, relu_rows),
        out_type=jax.ShapeDtypeStruct((NB, H), jnp.float32),
        mesh=plsc.VectorSubcoreMesh(**_SC_MESH),
        scratch_types=[
            pltpu.VMEM((_DBUF, _DSTEP), jnp.int32),
            pltpu.VMEM((_DBUF, _DSTEP), jnp.int32),
            pltpu.VMEM((_DBUF, _DSTEP, H), jnp.float32),
            pltpu.VMEM((_DBUF, _DSTEP, H), jnp.float32),
        ] + [pltpu.SemaphoreType.DMA] * 20,
    )()


_diff_relu = _make_diff(True)
_diff_plain = _make_diff(False)


_BM = 4000


def _proj_body(x_ref, w_ref, o_ref):
    o_ref[...] = jnp.dot(x_ref[...], w_ref[...],
                         preferred_element_type=jnp.float32)


def _proj(x, w):
    m, k = x.shape
    n = w.shape[1]
    return pl.pallas_call(
        _proj_body,
        grid=(m // _BM,),
        in_specs=[
            pl.BlockSpec((_BM, k), lambda i: (i, 0)),
            pl.BlockSpec((k, n), lambda i: (0, 0)),
        ],
        out_specs=pl.BlockSpec((_BM, n), lambda i: (i, 0)),
        out_shape=jax.ShapeDtypeStruct((m, n), jnp.float32),
    )(x, w)


def _projh_body(relu_x, x_ref, w_ref, o_ref):
    x = x_ref[...]
    if relu_x:
        x = jnp.maximum(x, 0.0)
    o_ref[...] = jnp.dot(x, w_ref[...], preferred_element_type=jnp.float32)


def _projh(x, w, relu_x=False, block=None):
    m = x.shape[0]
    bm = block or _BM
    return pl.pallas_call(
        functools.partial(_projh_body, relu_x),
        grid=(m // bm,),
        in_specs=[
            pl.BlockSpec((bm, H), lambda i: (i, 0)),
            pl.BlockSpec((H, H), lambda i: (0, 0)),
        ],
        out_specs=pl.BlockSpec((bm, H), lambda i: (i, 0)),
        out_shape=jax.ShapeDtypeStruct((m, H), jnp.float32),
    )(x, w)


def _resid_body(d_ref, inp_ref, o_ref):
    o_ref[...] = jnp.maximum(inp_ref[...] + d_ref[...], 0.0)


def _resid(d, inp):
    return pl.pallas_call(
        _resid_body,
        grid=(NB // _BM,),
        in_specs=[
            pl.BlockSpec((_BM, H), lambda i: (i, 0)),
            pl.BlockSpec((_BM, H), lambda i: (i, 0)),
        ],
        out_specs=pl.BlockSpec((_BM, H), lambda i: (i, 0)),
        out_shape=jax.ShapeDtypeStruct((NB, H), jnp.float32),
    )(d, inp)


def _update_body(d_ref, inp_ref, w_ref, o_ref):
    o_ref[...] = jnp.maximum(
        inp_ref[...] + jnp.dot(d_ref[...], w_ref[...],
                               preferred_element_type=jnp.float32), 0.0)


def _update(d, inp, w_h):
    return pl.pallas_call(
        _update_body,
        grid=(NB // _BM,),
        in_specs=[
            pl.BlockSpec((_BM, H), lambda i: (i, 0)),
            pl.BlockSpec((_BM, H), lambda i: (i, 0)),
            pl.BlockSpec((H, H), lambda i: (0, 0)),
        ],
        out_specs=pl.BlockSpec((_BM, H), lambda i: (i, 0)),
        out_shape=jax.ShapeDtypeStruct((NB, H), jnp.float32),
    )(d, inp, w_h)


def _atomout_body(fa_ref, am_ref, w1_ref, w2_ref, b_ref, o_ref):
    acc = jnp.dot(fa_ref[...], w1_ref[...], preferred_element_type=jnp.float32)
    acc += jnp.dot(am_ref[...], w2_ref[...], preferred_element_type=jnp.float32)
    o_ref[...] = jnp.maximum(acc + b_ref[...], 0.0)


def _atom_out(f_atoms, a_msg_pad, w1, w2, b):
    return pl.pallas_call(
        _atomout_body,
        grid=(NA // 1000,),
        in_specs=[
            pl.BlockSpec((1000, H), lambda i: (i, 0)),
            pl.BlockSpec((1000, H), lambda i: (i, 0)),
            pl.BlockSpec((H, H), lambda i: (0, 0)),
            pl.BlockSpec((H, H), lambda i: (0, 0)),
            pl.BlockSpec((1, H), lambda i: (0, 0)),
        ],
        out_specs=pl.BlockSpec((1000, H), lambda i: (i, 0)),
        out_shape=jax.ShapeDtypeStruct((NA, H), jnp.float32),
    )(f_atoms, a_msg_pad, w1, w2, b)


def _readout_body(h2_ref, w_ref, deg_ref, o_ref):
    acc = w_ref[:, 0:1] * h2_ref[:, 0:H]
    wsum = w_ref[:, 0:1]
    for k in range(1, APM):
        acc += w_ref[:, k:k + 1] * h2_ref[:, k * H:(k + 1) * H]
        wsum += w_ref[:, k:k + 1]
    o_ref[...] = deg_ref[...] * acc / wsum


def _readout(h2, w_atoms2, deg2):
    return pl.pallas_call(
        _readout_body,
        in_specs=[
            pl.BlockSpec((NMOL, APM * H), lambda: (0, 0)),
            pl.BlockSpec((NMOL, APM), lambda: (0, 0)),
            pl.BlockSpec((NMOL, 1), lambda: (0, 0)),
        ],
        out_specs=pl.BlockSpec((NMOL, H), lambda: (0, 0)),
        out_shape=jax.ShapeDtypeStruct((NMOL, H), jnp.float32),
    )(h2, w_atoms2, deg2)


def kernel(f_atoms, f_bonds, w_atoms, w_bonds, degree_of_polym,
           a2b, b2a, b2revb, W_i, W_h, W_o_w, W_o_b):
    a2b_flat = jnp.concatenate(
        [a2b, jnp.zeros((NA_PAD - NA, MAXNB), jnp.int32)], axis=0).reshape(-1)
    wo1 = W_o_w[:H]
    wo2 = W_o_w[H:]
    bias2 = W_o_b.reshape(1, H)
    w_atoms2 = w_atoms.reshape(NMOL, APM)
    deg2 = degree_of_polym.reshape(NMOL, 1)

    inp = _proj(f_bonds, W_i)

    a1 = _agg_relu(inp, a2b_flat, w_bonds)
    d1 = _diff_relu(a1, inp, b2a, b2revb)
    msg2 = _update(d1, inp, W_h)

    a2 = _agg_plain(msg2, a2b_flat, w_bonds)
    d2 = _diff_plain(a2, msg2, b2a, b2revb)
    msg3 = _update(d2, inp, W_h)

    a3 = _agg_plain(msg3, a2b_flat, w_bonds)
    h = _atom_out(f_atoms, a3[:NA], wo1, wo2, bias2)
    h2 = h.reshape(NMOL, APM * H)
    return _readout(h2, w_atoms2, deg2)
